# baseline (speedup 1.0000x reference)
import jax
import jax.numpy as jnp
from jax import lax
from jax.experimental import pallas as pl
from jax.experimental.pallas import tpu as pltpu
from jax.experimental.pallas import tpu_sc as plsc

NUM_EDGES = 5000
EDGE_DIM = 64
PAD_DIM = 128
DIRECTION = 8
BATCH = 4096
LANES = 16

NC = 2
NS = 16
NW = NC * NS
BPW = BATCH // NW


def _sc_gather(last_obs, goal, mat_flat, link_pad, de_pad, dh_pad):
    mesh = plsc.VectorSubcoreMesh(core_axis_name="c", subcore_axis_name="s")

    def body(lo_hbm, gl_hbm, mat_hbm, link_hbm, de_hbm, dh_hbm,
             out_link, out_de, out_dh,
             lo_v, gl_v, idx_v, gd_v, linkr_v, der_v, dhr_v, sem):
        wid = lax.axis_index("s") * NC + lax.axis_index("c")
        base = wid * BPW
        pltpu.sync_copy(lo_hbm.at[pl.ds(base, BPW)], lo_v)
        pltpu.sync_copy(gl_hbm.at[pl.ds(base, BPW)], gl_v)
        for j in range(BPW // LANES):
            lo = lo_v[pl.ds(j * LANES, LANES)]
            gl = gl_v[pl.ds(j * LANES, LANES)]
            row = jnp.where(lo == 0, NUM_EDGES - 1, lo - 1)
            idx_v[pl.ds(j * LANES, LANES)] = row * NUM_EDGES + gl
        pltpu.async_copy(mat_hbm.at[idx_v], gd_v, sem).wait()
        for j in range(BPW // LANES):
            gd_v[pl.ds(j * LANES, LANES)] = gd_v[pl.ds(j * LANES, LANES)] + 1
        pltpu.async_copy(link_hbm.at[lo_v], linkr_v, sem).wait()
        pltpu.async_copy(de_hbm.at[gd_v], der_v, sem).wait()
        pltpu.async_copy(dh_hbm.at[gd_v], dhr_v, sem).wait()
        pltpu.sync_copy(linkr_v, out_link.at[pl.ds(base, BPW)])
        pltpu.sync_copy(der_v, out_de.at[pl.ds(base, BPW)])
        pltpu.sync_copy(dhr_v, out_dh.at[pl.ds(base, BPW)])

    f32 = jnp.float32
    out_type = (
        jax.ShapeDtypeStruct((BATCH, PAD_DIM), f32),
        jax.ShapeDtypeStruct((BATCH, PAD_DIM), f32),
        jax.ShapeDtypeStruct((BATCH, PAD_DIM), f32),
    )
    scratch = [
        pltpu.VMEM((BPW,), jnp.int32),
        pltpu.VMEM((BPW,), jnp.int32),
        pltpu.VMEM((BPW,), jnp.int32),
        pltpu.VMEM((BPW,), jnp.int32),
        pltpu.VMEM((BPW, PAD_DIM), f32),
        pltpu.VMEM((BPW, PAD_DIM), f32),
        pltpu.VMEM((BPW, PAD_DIM), f32),
        pltpu.SemaphoreType.DMA,
    ]
    return pl.kernel(body, out_type=out_type, mesh=mesh, scratch_types=scratch)(
        last_obs, goal, mat_flat, link_pad, de_pad, dh_pad)


_ROWS_BLK = 1000


def _tail_body(a_ref, h_ref, out_ref):
    a = a_ref[...]
    for i in range(DIRECTION):
        h = h_ref[i:i + 1, :]
        dot = jnp.sum(a * h, axis=1, keepdims=True)
        out_ref[i, :, :] = a - h * dot


def _proj_body(lr_ref, dh_ref, de_ref, out_ref, out_de_ref):
    lr = lr_ref[...]
    dh = dh_ref[...]
    s = jnp.sum(lr * dh, axis=1, keepdims=True)
    out_ref[...] = (lr - dh * s)[:, :EDGE_DIM]
    out_de_ref[...] = de_ref[:, :EDGE_DIM]


def kernel(inputs, directions, mask, goal, loc_dlabels_matrix, link_table,
           dir_emb_table, dir_hyper_table):
    last_obs = inputs[:, -1].astype(jnp.int32)
    goal = goal.astype(jnp.int32)
    mat_flat = loc_dlabels_matrix.reshape(-1)

    pad = ((0, 0), (0, PAD_DIM - EDGE_DIM))
    link_pad = jnp.pad(link_table, pad)
    de_pad = jnp.pad(dir_emb_table, pad)
    dh_pad = jnp.pad(dir_hyper_table, pad)

    all_tail = link_table[1:, :]
    hyper = dir_hyper_table[1:, :]
    tails = pl.pallas_call(
        _tail_body,
        grid=(NUM_EDGES // _ROWS_BLK,),
        in_specs=[
            pl.BlockSpec((_ROWS_BLK, EDGE_DIM), lambda i: (i, 0)),
            pl.BlockSpec((DIRECTION, EDGE_DIM), lambda i: (0, 0)),
        ],
        out_specs=pl.BlockSpec((DIRECTION, _ROWS_BLK, EDGE_DIM),
                               lambda i: (0, i, 0)),
        out_shape=jax.ShapeDtypeStruct((DIRECTION, NUM_EDGES, EDGE_DIM),
                                       jnp.float32),
    )(all_tail, hyper)

    link_rows, de_rows, dh_rows = _sc_gather(
        last_obs, goal, mat_flat, link_pad, de_pad, dh_pad)

    link_embs, direction_embs = pl.pallas_call(
        _proj_body,
        out_shape=(jax.ShapeDtypeStruct((BATCH, EDGE_DIM), jnp.float32),
                   jax.ShapeDtypeStruct((BATCH, EDGE_DIM), jnp.float32)),
    )(link_rows, dh_rows, de_rows)

    return (link_embs, direction_embs, tails)


# speedup vs baseline: 1.2541x; 1.2541x over previous
import jax
import jax.numpy as jnp
from jax import lax
from jax.experimental import pallas as pl
from jax.experimental.pallas import tpu as pltpu
from jax.experimental.pallas import tpu_sc as plsc

NUM_EDGES = 5000
EDGE_DIM = 64
PAD_DIM = 128
DIRECTION = 8
BATCH = 4096
LANES = 16

NC = 2
NS = 16
NW = NC * NS
BPW = BATCH // NW


def _sc_gather(last_obs, goal, mat_flat, link_pad):
    mesh = plsc.VectorSubcoreMesh(core_axis_name="c", subcore_axis_name="s")

    def body(lo_hbm, gl_hbm, mat_hbm, link_hbm,
             out_gd, out_link,
             lo_v, gl_v, idx_v, gd_v, linkr_v, sem_m, sem_l):
        wid = lax.axis_index("s") * NC + lax.axis_index("c")
        base = wid * BPW
        pltpu.sync_copy(lo_hbm.at[pl.ds(base, BPW)], lo_v)
        pltpu.sync_copy(gl_hbm.at[pl.ds(base, BPW)], gl_v)
        link_cp = pltpu.async_copy(link_hbm.at[lo_v], linkr_v, sem_l)
        for j in range(BPW // LANES):
            lo = lo_v[pl.ds(j * LANES, LANES)]
            gl = gl_v[pl.ds(j * LANES, LANES)]
            row = jnp.where(lo == 0, NUM_EDGES - 1, lo - 1)
            idx_v[pl.ds(j * LANES, LANES)] = row * NUM_EDGES + gl
        pltpu.async_copy(mat_hbm.at[idx_v], gd_v, sem_m).wait()
        pltpu.sync_copy(gd_v, out_gd.at[pl.ds(base, BPW)])
        link_cp.wait()
        pltpu.sync_copy(linkr_v, out_link.at[pl.ds(base, BPW)])

    out_type = (
        jax.ShapeDtypeStruct((BATCH,), jnp.int32),
        jax.ShapeDtypeStruct((BATCH, PAD_DIM), jnp.float32),
    )
    scratch = [
        pltpu.VMEM((BPW,), jnp.int32),
        pltpu.VMEM((BPW,), jnp.int32),
        pltpu.VMEM((BPW,), jnp.int32),
        pltpu.VMEM((BPW,), jnp.int32),
        pltpu.VMEM((BPW, PAD_DIM), jnp.float32),
        pltpu.SemaphoreType.DMA,
        pltpu.SemaphoreType.DMA,
    ]
    return pl.kernel(body, out_type=out_type, mesh=mesh, scratch_types=scratch)(
        last_obs, goal, mat_flat, link_pad)


_ROWS_BLK = 1000


def _tail_body(a_ref, h_ref, out_ref):
    a = a_ref[...]
    for i in range(DIRECTION):
        h = h_ref[i:i + 1, :]
        dot = jnp.sum(a * h, axis=1, keepdims=True)
        out_ref[i, :, :] = a - h * dot


def _proj_body(gd_ref, lr_ref, de8_ref, dh8_ref, out_link_ref, out_de_ref):
    gd = gd_ref[...]
    onehot = (gd == lax.broadcasted_iota(jnp.int32, (BATCH, DIRECTION), 1))
    onehot = onehot.astype(jnp.float32)
    dh = jax.lax.dot_general(onehot, dh8_ref[...],
                             (((1,), (0,)), ((), ())),
                             preferred_element_type=jnp.float32)
    de = jax.lax.dot_general(onehot, de8_ref[...],
                             (((1,), (0,)), ((), ())),
                             preferred_element_type=jnp.float32)
    lr = lr_ref[:, :EDGE_DIM]
    s = jnp.sum(lr * dh, axis=1, keepdims=True)
    out_link_ref[...] = lr - dh * s
    out_de_ref[...] = de


def kernel(inputs, directions, mask, goal, loc_dlabels_matrix, link_table,
           dir_emb_table, dir_hyper_table):
    last_obs = inputs[:, -1].astype(jnp.int32)
    goal = goal.astype(jnp.int32)
    mat_flat = loc_dlabels_matrix.reshape(-1)

    link_pad = jnp.pad(link_table, ((0, 0), (0, PAD_DIM - EDGE_DIM)))

    all_tail = link_table[1:, :]
    hyper = dir_hyper_table[1:, :]
    tails = pl.pallas_call(
        _tail_body,
        grid=(NUM_EDGES // _ROWS_BLK,),
        in_specs=[
            pl.BlockSpec((_ROWS_BLK, EDGE_DIM), lambda i: (i, 0)),
            pl.BlockSpec((DIRECTION, EDGE_DIM), lambda i: (0, 0)),
        ],
        out_specs=pl.BlockSpec((DIRECTION, _ROWS_BLK, EDGE_DIM),
                               lambda i: (0, i, 0)),
        out_shape=jax.ShapeDtypeStruct((DIRECTION, NUM_EDGES, EDGE_DIM),
                                       jnp.float32),
    )(all_tail, hyper)

    gd_raw, link_rows = _sc_gather(last_obs, goal, mat_flat, link_pad)

    link_embs, direction_embs = pl.pallas_call(
        _proj_body,
        out_shape=(jax.ShapeDtypeStruct((BATCH, EDGE_DIM), jnp.float32),
                   jax.ShapeDtypeStruct((BATCH, EDGE_DIM), jnp.float32)),
    )(gd_raw.reshape(BATCH, 1), link_rows, dir_emb_table[1:, :], hyper)

    return (link_embs, direction_embs, tails)
